# Initial kernel scaffold; baseline (speedup 1.0000x reference)
import jax
import jax.numpy as jnp
from jax.experimental import pallas as pl


def kernel(z, x_t, lattice, edge_index, dist, t, y, mask, params):
    raise NotImplementedError("write your pallas kernel here")


# speedup vs baseline: 2.2316x; 2.2316x over previous
import functools

import jax
import jax.numpy as jnp
import numpy as np
from jax import lax
from jax.experimental import pallas as pl
from jax.experimental.pallas import tpu as pltpu
from jax.experimental.pallas import tpu_sc as plsc

N = 10000
E = 160000
NODE, EDGE, TIME, COND, HID = 192, 64, 128, 128, 256
CUTOFF, RBF, MAXZ = 5.0, 32, 100

NBLK = 1000
EBLK = 2000
CH = 128
NCH = E // CH
NW = 32
ROWS_PER_TILE = N // 16


def _silu(x):
    return x * (1.0 / (1.0 + jnp.exp(-x)))


def _embed_body(z_ref, x_ref, emb_ref, cw1_ref, cb1_ref, cw2_ref, cb2_ref,
                mw1h_ref, h_ref, p_ref):
    z = z_ref[0, 0, :]
    oh = (z[:, None] == lax.broadcasted_iota(jnp.int32, (1, 128), 1)
          ).astype(jnp.float32)
    h_emb = jnp.dot(oh, emb_ref[...], preferred_element_type=jnp.float32)
    x = x_ref[...]
    hc = jnp.dot(_silu(jnp.dot(x, cw1_ref[...],
                               preferred_element_type=jnp.float32)
                       + cb1_ref[...]),
                 cw2_ref[...], preferred_element_type=jnp.float32) + cb2_ref[...]
    h = h_emb + hc
    h_ref[...] = h
    p_ref[...] = jnp.dot(h, mw1h_ref[...], preferred_element_type=jnp.float32)


_embed_call = pl.pallas_call(
    _embed_body,
    grid=(N // NBLK,),
    in_specs=[
        pl.BlockSpec((1, 1, NBLK), lambda i: (i, 0, 0)),
        pl.BlockSpec((NBLK, 8), lambda i: (i, 0)),
        pl.BlockSpec((128, NODE), lambda i: (0, 0)),
        pl.BlockSpec((8, NODE), lambda i: (0, 0)),
        pl.BlockSpec((1, NODE), lambda i: (0, 0)),
        pl.BlockSpec((NODE, NODE), lambda i: (0, 0)),
        pl.BlockSpec((1, NODE), lambda i: (0, 0)),
        pl.BlockSpec((NODE, HID), lambda i: (0, 0)),
    ],
    out_specs=[
        pl.BlockSpec((NBLK, NODE), lambda i: (i, 0)),
        pl.BlockSpec((NBLK, HID), lambda i: (i, 0)),
    ],
    out_shape=[
        jax.ShapeDtypeStruct((N, NODE), jnp.float32),
        jax.ShapeDtypeStruct((N, HID), jnp.float32),
    ],
)


def _edge_body(d_ref, g_ref, ew1a_ref, ew1b_ref, eb1_ref, ew2_ref, eb2_ref,
               e_ref):
    d = jnp.clip(d_ref[0, 0, :], 0.0, CUTOFF)
    dc = d[:, None]
    centers = (lax.broadcasted_iota(jnp.int32, (1, RBF), 1)
               .astype(jnp.float32) * (CUTOFF / (RBF - 1)))
    rbf = jnp.exp(-g_ref[0, 0] * (dc - centers) ** 2)
    f1 = (jnp.dot(rbf, ew1a_ref[...], preferred_element_type=jnp.float32)
          + (dc / CUTOFF) * ew1b_ref[...] + eb1_ref[...])
    e_ref[...] = (jnp.dot(_silu(f1), ew2_ref[...],
                          preferred_element_type=jnp.float32) + eb2_ref[...])


_edge_call = pl.pallas_call(
    _edge_body,
    grid=(E // EBLK,),
    in_specs=[
        pl.BlockSpec((1, 1, EBLK), lambda i: (i, 0, 0)),
        pl.BlockSpec((1, 1), lambda i: (0, 0)),
        pl.BlockSpec((RBF, EDGE), lambda i: (0, 0)),
        pl.BlockSpec((1, EDGE), lambda i: (0, 0)),
        pl.BlockSpec((1, EDGE), lambda i: (0, 0)),
        pl.BlockSpec((EDGE, EDGE), lambda i: (0, 0)),
        pl.BlockSpec((1, EDGE), lambda i: (0, 0)),
    ],
    out_specs=pl.BlockSpec((EBLK, EDGE), lambda i: (i, 0)),
    out_shape=jax.ShapeDtypeStruct((E, EDGE), jnp.float32),
)


def _msg_body(ps_ref, e_ref, mw1e_ref, cm_ref, mw2_ref, mb2_ref,
              lo_ref, hi_ref):
    z1 = (ps_ref[...]
          + jnp.dot(e_ref[...], mw1e_ref[...],
                    preferred_element_type=jnp.float32)
          + cm_ref[...])
    msg = _silu(jnp.dot(_silu(z1), mw2_ref[...],
                        preferred_element_type=jnp.float32) + mb2_ref[...])
    lo_ref[...] = msg[:, :128]
    hi_ref[...] = msg[:, 128:]


_msg_call = pl.pallas_call(
    _msg_body,
    grid=(E // EBLK,),
    in_specs=[
        pl.BlockSpec((EBLK, HID), lambda i: (i, 0)),
        pl.BlockSpec((EBLK, EDGE), lambda i: (i, 0)),
        pl.BlockSpec((EDGE, HID), lambda i: (0, 0)),
        pl.BlockSpec((1, HID), lambda i: (0, 0)),
        pl.BlockSpec((HID, HID), lambda i: (0, 0)),
        pl.BlockSpec((1, HID), lambda i: (0, 0)),
    ],
    out_specs=[
        pl.BlockSpec((EBLK, 128), lambda i: (i, 0)),
        pl.BlockSpec((EBLK, 128), lambda i: (i, 0)),
    ],
    out_shape=[
        jax.ShapeDtypeStruct((E, 128), jnp.float32),
        jax.ShapeDtypeStruct((E, 128), jnp.float32),
    ],
)


def _node_body(h_ref, al_ref, ah_ref, uw1h_ref, uw1a_ref, cu_ref, uw2_ref,
               ub2_ref, g_ref, b_ref, nxt_ref, h_out_ref, p_out_ref,
               *, last):
    h = h_ref[...]
    agg = jnp.concatenate([al_ref[...], ah_ref[...]], axis=1)
    u1 = (jnp.dot(h, uw1h_ref[...], preferred_element_type=jnp.float32)
          + jnp.dot(agg, uw1a_ref[...], preferred_element_type=jnp.float32)
          + cu_ref[...])
    dh = jnp.dot(_silu(u1), uw2_ref[...],
                 preferred_element_type=jnp.float32) + ub2_ref[...]
    x = h + dh
    mu = jnp.mean(x, axis=-1, keepdims=True)
    xm = x - mu
    var = jnp.mean(xm * xm, axis=-1, keepdims=True)
    hn = xm * lax.rsqrt(var + 1e-5) * g_ref[...] + b_ref[...]
    h_out_ref[...] = hn
    if last:
        ow1h = nxt_ref[0:NODE, :]
        co = nxt_ref[NODE:NODE + 1, :]
        ow2 = nxt_ref[NODE + 1:NODE + 1 + HID, :]
        ob2 = nxt_ref[NODE + 1 + HID:NODE + 2 + HID, :]
        ow3 = nxt_ref[NODE + 2 + HID:NODE + 2 + 2 * HID, 0:8]
        ob3 = nxt_ref[NODE + 2 + 2 * HID:NODE + 3 + 2 * HID, 0:8]
        a1 = _silu(jnp.dot(hn, ow1h, preferred_element_type=jnp.float32) + co)
        a2 = _silu(jnp.dot(a1, ow2, preferred_element_type=jnp.float32) + ob2)
        p_out_ref[...] = jnp.dot(a2, ow3,
                                 preferred_element_type=jnp.float32) + ob3
    else:
        p_out_ref[...] = jnp.dot(hn, nxt_ref[...],
                                 preferred_element_type=jnp.float32)


def _make_node_call(last):
    nxt_shape = (NODE + 2 * HID + 3, HID) if last else (NODE, HID)
    p_shape = (N, 8) if last else (N, HID)
    p_blk = (NBLK, 8) if last else (NBLK, HID)
    return pl.pallas_call(
        functools.partial(_node_body, last=last),
        grid=(N // NBLK,),
        in_specs=[
            pl.BlockSpec((NBLK, NODE), lambda i: (i, 0)),
            pl.BlockSpec((NBLK, 128), lambda i: (i, 0)),
            pl.BlockSpec((NBLK, 128), lambda i: (i, 0)),
            pl.BlockSpec((NODE, HID), lambda i: (0, 0)),
            pl.BlockSpec((HID, HID), lambda i: (0, 0)),
            pl.BlockSpec((1, HID), lambda i: (0, 0)),
            pl.BlockSpec((HID, NODE), lambda i: (0, 0)),
            pl.BlockSpec((1, NODE), lambda i: (0, 0)),
            pl.BlockSpec((1, NODE), lambda i: (0, 0)),
            pl.BlockSpec((1, NODE), lambda i: (0, 0)),
            pl.BlockSpec(nxt_shape, lambda i: (0, 0)),
        ],
        out_specs=[
            pl.BlockSpec((NBLK, NODE), lambda i: (i, 0)),
            pl.BlockSpec(p_blk, lambda i: (i, 0)),
        ],
        out_shape=[
            jax.ShapeDtypeStruct((N, NODE), jnp.float32),
            jax.ShapeDtypeStruct(p_shape, jnp.float32),
        ],
    )


_node_call_mid = _make_node_call(False)
_node_call_last = _make_node_call(True)


_SC_MESH = plsc.VectorSubcoreMesh(core_axis_name="c", subcore_axis_name="s")
_GITER = -(-NCH // NW)


def _gather_body(p_hbm, src_hbm, out_hbm, idx_v, rows_v, sem):
    wid = lax.axis_index("s") * 2 + lax.axis_index("c")

    def body(i, _):
        cid = wid + i * NW

        @pl.when(cid < NCH)
        def _():
            base = cid * CH
            pltpu.sync_copy(src_hbm.at[pl.ds(base, CH)], idx_v)
            pltpu.async_copy(p_hbm.at[idx_v], rows_v, sem).wait()
            pltpu.sync_copy(rows_v, out_hbm.at[pl.ds(base, CH)])
        return 0

    lax.fori_loop(0, _GITER, body, 0)


_gather_call = pl.kernel(
    _gather_body,
    out_type=jax.ShapeDtypeStruct((E, HID), jnp.float32),
    mesh=_SC_MESH,
    scratch_types=[
        pltpu.VMEM((CH,), jnp.int32),
        pltpu.VMEM((CH, HID), jnp.float32),
        pltpu.SemaphoreType.DMA,
    ],
)


_SITER = -(-NCH // 16)


def _scatter_body(ml_hbm, mh_hbm, dst_hbm, al_hbm, ah_hbm,
                  idx_v, buf_v, zb_v, acc_s):
    c = lax.axis_index("c")
    s = lax.axis_index("s")

    def zrow(i, _):
        for kk in range(8):
            zb_v[i, pl.ds(kk * 16, 16)] = jnp.zeros((16,), jnp.float32)
        return 0

    lax.fori_loop(0, 128, zrow, 0)
    row0 = s * 624
    nfull = jnp.where(s == 15, 5, 4)

    def zcopy(i, _):
        pltpu.sync_copy(zb_v, acc_s.at[pl.ds(row0 + i * 128, 128)])
        return 0

    lax.fori_loop(0, nfull, zcopy, 0)

    @pl.when(s < 15)
    def _():
        pltpu.sync_copy(zb_v.at[pl.ds(0, 112)],
                        acc_s.at[pl.ds(row0 + 512, 112)])

    plsc.subcore_barrier()

    def body(i, _):
        cid = s + i * 16

        @pl.when(cid < NCH)
        def _():
            base = cid * CH
            pltpu.sync_copy(dst_hbm.at[pl.ds(base, CH)], idx_v)

            @pl.when(c == 0)
            def _():
                pltpu.sync_copy(ml_hbm.at[pl.ds(base, CH)], buf_v)

            @pl.when(c == 1)
            def _():
                pltpu.sync_copy(mh_hbm.at[pl.ds(base, CH)], buf_v)

            pltpu.sync_copy(buf_v, acc_s.at[idx_v], add=True)
        return 0

    lax.fori_loop(0, _SITER, body, 0)
    plsc.subcore_barrier()

    @pl.when((c == 0) & (s < 15))
    def _():
        pltpu.sync_copy(acc_s.at[pl.ds(row0, 624)],
                        al_hbm.at[pl.ds(row0, 624)])

    @pl.when((c == 1) & (s < 15))
    def _():
        pltpu.sync_copy(acc_s.at[pl.ds(row0, 624)],
                        ah_hbm.at[pl.ds(row0, 624)])

    @pl.when((c == 0) & (s == 15))
    def _():
        pltpu.sync_copy(acc_s.at[pl.ds(15 * 624, 640)],
                        al_hbm.at[pl.ds(15 * 624, 640)])

    @pl.when((c == 1) & (s == 15))
    def _():
        pltpu.sync_copy(acc_s.at[pl.ds(15 * 624, 640)],
                        ah_hbm.at[pl.ds(15 * 624, 640)])


_scatter_call = pl.kernel(
    _scatter_body,
    out_type=[
        jax.ShapeDtypeStruct((N, 128), jnp.float32),
        jax.ShapeDtypeStruct((N, 128), jnp.float32),
    ],
    mesh=_SC_MESH,
    scratch_types=[
        pltpu.VMEM((CH,), jnp.int32),
        pltpu.VMEM((CH, 128), jnp.float32),
        pltpu.VMEM((128, 128), jnp.float32),
        pltpu.VMEM_SHARED((N, 128), jnp.float32),
    ],
)


def kernel(z, x_t, lattice, edge_index, dist, t, y, mask, params):
    p = params
    half = TIME // 2
    freqs = jnp.exp(jnp.linspace(0.0, np.log(10000.0), half) * -1.0)
    targ = t[:, None] * freqs[None, :]
    temb = jnp.concatenate([jnp.sin(targ), jnp.cos(targ)], axis=-1)
    tp = p['time']
    t_emb = _silu(temb @ tp['W1'] + tp['b1']) @ tp['W2'] + tp['b2']
    pp = p['prop']
    y_emb = (_silu(_silu(y @ pp['W1'] + pp['b1']) @ pp['W2'] + pp['b2'])
             @ pp['W3'] + pp['b3'])
    cond = t_emb + y_emb

    emb_pad = jnp.zeros((128, NODE), jnp.float32).at[:MAXZ + 1].set(
        p['atom_emb'])
    c = p['coord']
    cw1 = jnp.zeros((8, NODE), jnp.float32).at[:2].set(c['W1'][:2]).at[2].set(
        c['W1'][2] * 0.1)
    ep = p['edge']
    gam = ep['gamma'].reshape(1, 1)
    ew1a, ew1b = ep['W1'][:RBF], ep['W1'][RBF:RBF + 1]

    gnn = p['gnn']
    cm = [cond @ lp['mW1'][NODE + EDGE:] + lp['mb1'][None] for lp in gnn]
    cu = [cond @ lp['uW1'][NODE + HID:] + lp['ub1'][None] for lp in gnn]
    op = p['out']
    co = cond @ op['W1'][NODE:] + op['b1'][None]
    ow3 = jnp.zeros((HID, 8), jnp.float32).at[:, :3].set(op['W3'])
    ob3 = jnp.zeros((8,), jnp.float32).at[:3].set(op['b3'])
    pack_last = jnp.concatenate([
        op['W1'][:NODE],
        co,
        op['W2'],
        op['b2'][None],
        jnp.zeros((HID, HID), jnp.float32).at[:, :8].set(ow3),
        jnp.zeros((1, HID), jnp.float32).at[0, :8].set(ob3),
    ], axis=0)

    z3 = z.reshape(N // NBLK, 1, NBLK).astype(jnp.int32)
    xp = jnp.zeros((N, 8), jnp.float32).at[:, :3].set(x_t[0])
    d3 = dist.reshape(E // EBLK, 1, EBLK)
    src = edge_index[0].astype(jnp.int32)
    dst = edge_index[1].astype(jnp.int32)

    h, pcur = _embed_call(z3, xp, emb_pad, cw1, c['b1'][None], c['W2'],
                          c['b2'][None], gnn[0]['mW1'][:NODE])
    e = _edge_call(d3, gam, ew1a, ew1b, ep['b1'][None], ep['W2'],
                   ep['b2'][None])

    for l in range(3):
        lp = gnn[l]
        p_src = _gather_call(pcur, src)
        mlo, mhi = _msg_call(p_src, e, lp['mW1'][NODE:NODE + EDGE], cm[l],
                             lp['mW2'], lp['mb2'][None])
        aggl, aggh = _scatter_call(mlo, mhi, dst)
        if l < 2:
            nxt = gnn[l + 1]['mW1'][:NODE]
            call = _node_call_mid
        else:
            nxt = pack_last
            call = _node_call_last
        h, pcur = call(h, aggl, aggh, lp['uW1'][:NODE],
                       lp['uW1'][NODE:NODE + HID], cu[l], lp['uW2'],
                       lp['ub2'][None], lp['ln_g'][None], lp['ln_b'][None],
                       nxt)

    return pcur[:, :3].reshape(1, N, 3)


# speedup vs baseline: 3.1898x; 1.4294x over previous
import functools

import jax
import jax.numpy as jnp
import numpy as np
from jax import lax
from jax.experimental import pallas as pl
from jax.experimental.pallas import tpu as pltpu
from jax.experimental.pallas import tpu_sc as plsc

N = 10000
E = 160000
NODE, EDGE, TIME, COND, HID = 192, 64, 128, 128, 256
CUTOFF, RBF, MAXZ = 5.0, 32, 100

NBLK = 1000
EBLK = 2000
CH = 128
NCH = E // CH
NW = 32
ROWS_PER_TILE = N // 16


def _silu(x):
    return x * (1.0 / (1.0 + jnp.exp(-x)))


def _pack_bf16(x):
    lo = lax.bitcast_convert_type(x[:, :128].astype(jnp.bfloat16),
                                  jnp.uint16).astype(jnp.uint32)
    hi = lax.bitcast_convert_type(x[:, 128:].astype(jnp.bfloat16),
                                  jnp.uint16).astype(jnp.uint32)
    return lo | (hi << 16)


def _unpack_bf16(u):
    lo = lax.bitcast_convert_type((u & 0xFFFF).astype(jnp.uint16),
                                  jnp.bfloat16).astype(jnp.float32)
    hi = lax.bitcast_convert_type((u >> 16).astype(jnp.uint16),
                                  jnp.bfloat16).astype(jnp.float32)
    return jnp.concatenate([lo, hi], axis=1)


def _embed_body(z_ref, x_ref, emb_ref, cw1_ref, cb1_ref, cw2_ref, cb2_ref,
                mw1h_ref, h_ref, p_ref):
    z = z_ref[0, 0, :]
    oh = (z[:, None] == lax.broadcasted_iota(jnp.int32, (1, 128), 1)
          ).astype(jnp.float32)
    h_emb = jnp.dot(oh, emb_ref[...], preferred_element_type=jnp.float32)
    x = x_ref[...]
    hc = jnp.dot(_silu(jnp.dot(x, cw1_ref[...],
                               preferred_element_type=jnp.float32)
                       + cb1_ref[...]),
                 cw2_ref[...], preferred_element_type=jnp.float32) + cb2_ref[...]
    h = h_emb + hc
    h_ref[...] = h
    p_ref[...] = _pack_bf16(
        jnp.dot(h, mw1h_ref[...], preferred_element_type=jnp.float32))


_embed_call = pl.pallas_call(
    _embed_body,
    grid=(N // NBLK,),
    in_specs=[
        pl.BlockSpec((1, 1, NBLK), lambda i: (i, 0, 0)),
        pl.BlockSpec((NBLK, 8), lambda i: (i, 0)),
        pl.BlockSpec((128, NODE), lambda i: (0, 0)),
        pl.BlockSpec((8, NODE), lambda i: (0, 0)),
        pl.BlockSpec((1, NODE), lambda i: (0, 0)),
        pl.BlockSpec((NODE, NODE), lambda i: (0, 0)),
        pl.BlockSpec((1, NODE), lambda i: (0, 0)),
        pl.BlockSpec((NODE, HID), lambda i: (0, 0)),
    ],
    out_specs=[
        pl.BlockSpec((NBLK, NODE), lambda i: (i, 0)),
        pl.BlockSpec((NBLK, 128), lambda i: (i, 0)),
    ],
    out_shape=[
        jax.ShapeDtypeStruct((N, NODE), jnp.float32),
        jax.ShapeDtypeStruct((N, 128), jnp.uint32),
    ],
)


def _edge_body(d_ref, g_ref, ew1a_ref, ew1b_ref, eb1_ref, ew2_ref, eb2_ref,
               e_ref):
    d = jnp.clip(d_ref[0, 0, :], 0.0, CUTOFF)
    dc = d[:, None]
    centers = (lax.broadcasted_iota(jnp.int32, (1, RBF), 1)
               .astype(jnp.float32) * (CUTOFF / (RBF - 1)))
    rbf = jnp.exp(-g_ref[0, 0] * (dc - centers) ** 2)
    f1 = (jnp.dot(rbf, ew1a_ref[...], preferred_element_type=jnp.float32)
          + (dc / CUTOFF) * ew1b_ref[...] + eb1_ref[...])
    e_ref[...] = (jnp.dot(_silu(f1), ew2_ref[...],
                          preferred_element_type=jnp.float32) + eb2_ref[...])


_edge_call = pl.pallas_call(
    _edge_body,
    grid=(E // EBLK,),
    in_specs=[
        pl.BlockSpec((1, 1, EBLK), lambda i: (i, 0, 0)),
        pl.BlockSpec((1, 1), lambda i: (0, 0)),
        pl.BlockSpec((RBF, EDGE), lambda i: (0, 0)),
        pl.BlockSpec((1, EDGE), lambda i: (0, 0)),
        pl.BlockSpec((1, EDGE), lambda i: (0, 0)),
        pl.BlockSpec((EDGE, EDGE), lambda i: (0, 0)),
        pl.BlockSpec((1, EDGE), lambda i: (0, 0)),
    ],
    out_specs=pl.BlockSpec((EBLK, EDGE), lambda i: (i, 0)),
    out_shape=jax.ShapeDtypeStruct((E, EDGE), jnp.float32),
)


def _msg_body(ps_ref, e_ref, mw1e_ref, cm_ref, mw2_ref, mb2_ref,
              lo_ref, hi_ref):
    z1 = (_unpack_bf16(ps_ref[...])
          + jnp.dot(e_ref[...], mw1e_ref[...],
                    preferred_element_type=jnp.float32)
          + cm_ref[...])
    msg = _silu(jnp.dot(_silu(z1), mw2_ref[...],
                        preferred_element_type=jnp.float32) + mb2_ref[...])
    lo_ref[...] = msg[:, :128]
    hi_ref[...] = msg[:, 128:]


_msg_call = pl.pallas_call(
    _msg_body,
    grid=(E // EBLK,),
    in_specs=[
        pl.BlockSpec((EBLK, 128), lambda i: (i, 0)),
        pl.BlockSpec((EBLK, EDGE), lambda i: (i, 0)),
        pl.BlockSpec((EDGE, HID), lambda i: (0, 0)),
        pl.BlockSpec((1, HID), lambda i: (0, 0)),
        pl.BlockSpec((HID, HID), lambda i: (0, 0)),
        pl.BlockSpec((1, HID), lambda i: (0, 0)),
    ],
    out_specs=[
        pl.BlockSpec((EBLK, 128), lambda i: (i, 0)),
        pl.BlockSpec((EBLK, 128), lambda i: (i, 0)),
    ],
    out_shape=[
        jax.ShapeDtypeStruct((E, 128), jnp.float32),
        jax.ShapeDtypeStruct((E, 128), jnp.float32),
    ],
)


def _node_body(h_ref, al_ref, ah_ref, uw1h_ref, uw1a_ref, cu_ref, uw2_ref,
               ub2_ref, g_ref, b_ref, nxt_ref, h_out_ref, p_out_ref,
               *, last):
    h = h_ref[...]
    agg = jnp.concatenate([al_ref[...], ah_ref[...]], axis=1)
    u1 = (jnp.dot(h, uw1h_ref[...], preferred_element_type=jnp.float32)
          + jnp.dot(agg, uw1a_ref[...], preferred_element_type=jnp.float32)
          + cu_ref[...])
    dh = jnp.dot(_silu(u1), uw2_ref[...],
                 preferred_element_type=jnp.float32) + ub2_ref[...]
    x = h + dh
    mu = jnp.mean(x, axis=-1, keepdims=True)
    xm = x - mu
    var = jnp.mean(xm * xm, axis=-1, keepdims=True)
    hn = xm * lax.rsqrt(var + 1e-5) * g_ref[...] + b_ref[...]
    h_out_ref[...] = hn
    if last:
        ow1h = nxt_ref[0:NODE, :]
        co = nxt_ref[NODE:NODE + 1, :]
        ow2 = nxt_ref[NODE + 1:NODE + 1 + HID, :]
        ob2 = nxt_ref[NODE + 1 + HID:NODE + 2 + HID, :]
        ow3 = nxt_ref[NODE + 2 + HID:NODE + 2 + 2 * HID, 0:8]
        ob3 = nxt_ref[NODE + 2 + 2 * HID:NODE + 3 + 2 * HID, 0:8]
        a1 = _silu(jnp.dot(hn, ow1h, preferred_element_type=jnp.float32) + co)
        a2 = _silu(jnp.dot(a1, ow2, preferred_element_type=jnp.float32) + ob2)
        p_out_ref[...] = jnp.dot(a2, ow3,
                                 preferred_element_type=jnp.float32) + ob3
    else:
        p_out_ref[...] = _pack_bf16(jnp.dot(hn, nxt_ref[...],
                                            preferred_element_type=jnp.float32))


def _make_node_call(last):
    nxt_shape = (NODE + 2 * HID + 3, HID) if last else (NODE, HID)
    p_shape = (N, 8) if last else (N, 128)
    p_blk = (NBLK, 8) if last else (NBLK, 128)
    p_dtype = jnp.float32 if last else jnp.uint32
    return pl.pallas_call(
        functools.partial(_node_body, last=last),
        grid=(N // NBLK,),
        in_specs=[
            pl.BlockSpec((NBLK, NODE), lambda i: (i, 0)),
            pl.BlockSpec((NBLK, 128), lambda i: (i, 0)),
            pl.BlockSpec((NBLK, 128), lambda i: (i, 0)),
            pl.BlockSpec((NODE, HID), lambda i: (0, 0)),
            pl.BlockSpec((HID, HID), lambda i: (0, 0)),
            pl.BlockSpec((1, HID), lambda i: (0, 0)),
            pl.BlockSpec((HID, NODE), lambda i: (0, 0)),
            pl.BlockSpec((1, NODE), lambda i: (0, 0)),
            pl.BlockSpec((1, NODE), lambda i: (0, 0)),
            pl.BlockSpec((1, NODE), lambda i: (0, 0)),
            pl.BlockSpec(nxt_shape, lambda i: (0, 0)),
        ],
        out_specs=[
            pl.BlockSpec((NBLK, NODE), lambda i: (i, 0)),
            pl.BlockSpec(p_blk, lambda i: (i, 0)),
        ],
        out_shape=[
            jax.ShapeDtypeStruct((N, NODE), jnp.float32),
            jax.ShapeDtypeStruct(p_shape, p_dtype),
        ],
    )


_node_call_mid = _make_node_call(False)
_node_call_last = _make_node_call(True)


_SC_MESH = plsc.VectorSubcoreMesh(core_axis_name="c", subcore_axis_name="s")
_EPT = E // NW
_GFULL = _EPT // CH
_GTAIL = _EPT - _GFULL * CH


def _gather_body(p_hbm, src_hbm, out_hbm, idx_v, r0, r1, tail_v,
                 sg0, sg1, sw0, sw1):
    wid = lax.axis_index("s") * 2 + lax.axis_index("c")
    base = wid * _EPT
    pltpu.sync_copy(src_hbm.at[pl.ds(base, _EPT)], idx_v)
    rows = (r0, r1)
    sg = (sg0, sg1)
    sw = (sw0, sw1)

    def g_dma(k, b):
        return pltpu.make_async_copy(
            p_hbm.at[idx_v.at[pl.ds(k * CH, CH)]], rows[b], sg[b])

    def w_dma(k, b):
        return pltpu.make_async_copy(
            rows[b], out_hbm.at[pl.ds(base + k * CH, CH)], sw[b])

    g_dma(0, 0).start()

    def pair(i, _):
        for par in (0, 1):
            k = 2 * i + par

            @pl.when(k < _GFULL)
            def _():
                g_dma(k, par).wait()
                w_dma(k, par).start()

                @pl.when(k >= 1)
                def _():
                    w_dma(k - 1, 1 - par).wait()

                @pl.when(k + 1 < _GFULL)
                def _():
                    g_dma(k + 1, 1 - par).start()
        return 0

    lax.fori_loop(0, (_GFULL + 1) // 2, pair, 0)
    w_dma(_GFULL - 1, (_GFULL - 1) % 2).wait()
    pltpu.async_copy(p_hbm.at[idx_v.at[pl.ds(_GFULL * CH, _GTAIL)]],
                     tail_v, sg0).wait()
    pltpu.sync_copy(tail_v, out_hbm.at[pl.ds(base + _GFULL * CH, _GTAIL)])


_gather_call = pl.kernel(
    _gather_body,
    out_type=jax.ShapeDtypeStruct((E, 128), jnp.uint32),
    mesh=_SC_MESH,
    scratch_types=[
        pltpu.VMEM((_EPT,), jnp.int32),
        pltpu.VMEM((CH, 128), jnp.uint32),
        pltpu.VMEM((CH, 128), jnp.uint32),
        pltpu.VMEM((_GTAIL, 128), jnp.uint32),
        pltpu.SemaphoreType.DMA,
        pltpu.SemaphoreType.DMA,
        pltpu.SemaphoreType.DMA,
        pltpu.SemaphoreType.DMA,
    ],
)


_SITER = -(-NCH // 16)


def _scatter_body(ml_hbm, mh_hbm, dst_hbm, al_hbm, ah_hbm,
                  i0_v, i1_v, b0_v, b1_v, zb_v, acc_s,
                  si0, si1, sm0, sm1, ss0, ss1):
    c = lax.axis_index("c")
    s = lax.axis_index("s")

    def zrow(i, _):
        for kk in range(8):
            zb_v[i, pl.ds(kk * 16, 16)] = jnp.zeros((16,), jnp.float32)
        return 0

    lax.fori_loop(0, 128, zrow, 0)
    row0 = s * 624
    nfull = jnp.where(s == 15, 5, 4)

    def zcopy(i, _):
        pltpu.sync_copy(zb_v, acc_s.at[pl.ds(row0 + i * 128, 128)])
        return 0

    lax.fori_loop(0, nfull, zcopy, 0)

    @pl.when(s < 15)
    def _():
        pltpu.sync_copy(zb_v.at[pl.ds(0, 112)],
                        acc_s.at[pl.ds(row0 + 512, 112)])

    plsc.subcore_barrier()

    idx = (i0_v, i1_v)
    buf = (b0_v, b1_v)
    si = (si0, si1)
    sm = (sm0, sm1)
    ss = (ss0, ss1)

    def valid(k):
        return (s + k * 16) < NCH

    def reads_start(k, b):
        base = (s + k * 16) * CH
        pltpu.make_async_copy(dst_hbm.at[pl.ds(base, CH)], idx[b],
                              si[b]).start()

        @pl.when(c == 0)
        def _():
            pltpu.make_async_copy(ml_hbm.at[pl.ds(base, CH)], buf[b],
                                  sm[b]).start()

        @pl.when(c == 1)
        def _():
            pltpu.make_async_copy(mh_hbm.at[pl.ds(base, CH)], buf[b],
                                  sm[b]).start()

    def reads_wait(k, b):
        base = (s + k * 16) * CH
        pltpu.make_async_copy(dst_hbm.at[pl.ds(base, CH)], idx[b],
                              si[b]).wait()
        pltpu.make_async_copy(ml_hbm.at[pl.ds(base, CH)], buf[b],
                              sm[b]).wait()

    def sc_dma(b):
        return pltpu.make_async_copy(buf[b], acc_s.at[idx[b]], ss[b])

    @pl.when(valid(0))
    def _():
        reads_start(0, 0)

    def pair(i, _):
        for par in (0, 1):
            k = 2 * i + par

            @pl.when(valid(k))
            def _():
                reads_wait(k, par)
                sc_dma(par).start(add=True)

                @pl.when(k >= 1)
                def _():
                    sc_dma(1 - par).wait()

                @pl.when(valid(k + 1))
                def _():
                    reads_start(k + 1, 1 - par)
        return 0

    lax.fori_loop(0, (_SITER + 1) // 2, pair, 0)
    for k in (_SITER - 2, _SITER - 1):
        @pl.when(valid(k) & ~valid(k + 1))
        def _():
            sc_dma(k % 2).wait()

    plsc.subcore_barrier()

    @pl.when((c == 0) & (s < 15))
    def _():
        pltpu.sync_copy(acc_s.at[pl.ds(row0, 624)],
                        al_hbm.at[pl.ds(row0, 624)])

    @pl.when((c == 1) & (s < 15))
    def _():
        pltpu.sync_copy(acc_s.at[pl.ds(row0, 624)],
                        ah_hbm.at[pl.ds(row0, 624)])

    @pl.when((c == 0) & (s == 15))
    def _():
        pltpu.sync_copy(acc_s.at[pl.ds(15 * 624, 640)],
                        al_hbm.at[pl.ds(15 * 624, 640)])

    @pl.when((c == 1) & (s == 15))
    def _():
        pltpu.sync_copy(acc_s.at[pl.ds(15 * 624, 640)],
                        ah_hbm.at[pl.ds(15 * 624, 640)])


_scatter_call = pl.kernel(
    _scatter_body,
    out_type=[
        jax.ShapeDtypeStruct((N, 128), jnp.float32),
        jax.ShapeDtypeStruct((N, 128), jnp.float32),
    ],
    mesh=_SC_MESH,
    scratch_types=[
        pltpu.VMEM((CH,), jnp.int32),
        pltpu.VMEM((CH,), jnp.int32),
        pltpu.VMEM((CH, 128), jnp.float32),
        pltpu.VMEM((CH, 128), jnp.float32),
        pltpu.VMEM((128, 128), jnp.float32),
        pltpu.VMEM_SHARED((N, 128), jnp.float32),
        pltpu.SemaphoreType.DMA,
        pltpu.SemaphoreType.DMA,
        pltpu.SemaphoreType.DMA,
        pltpu.SemaphoreType.DMA,
        pltpu.SemaphoreType.DMA,
        pltpu.SemaphoreType.DMA,
    ],
)


def kernel(z, x_t, lattice, edge_index, dist, t, y, mask, params):
    p = params
    half = TIME // 2
    freqs = jnp.exp(jnp.linspace(0.0, np.log(10000.0), half) * -1.0)
    targ = t[:, None] * freqs[None, :]
    temb = jnp.concatenate([jnp.sin(targ), jnp.cos(targ)], axis=-1)
    tp = p['time']
    t_emb = _silu(temb @ tp['W1'] + tp['b1']) @ tp['W2'] + tp['b2']
    pp = p['prop']
    y_emb = (_silu(_silu(y @ pp['W1'] + pp['b1']) @ pp['W2'] + pp['b2'])
             @ pp['W3'] + pp['b3'])
    cond = t_emb + y_emb

    emb_pad = jnp.zeros((128, NODE), jnp.float32).at[:MAXZ + 1].set(
        p['atom_emb'])
    c = p['coord']
    cw1 = jnp.zeros((8, NODE), jnp.float32).at[:2].set(c['W1'][:2]).at[2].set(
        c['W1'][2] * 0.1)
    ep = p['edge']
    gam = ep['gamma'].reshape(1, 1)
    ew1a, ew1b = ep['W1'][:RBF], ep['W1'][RBF:RBF + 1]

    gnn = p['gnn']
    cm = [cond @ lp['mW1'][NODE + EDGE:] + lp['mb1'][None] for lp in gnn]
    cu = [cond @ lp['uW1'][NODE + HID:] + lp['ub1'][None] for lp in gnn]
    op = p['out']
    co = cond @ op['W1'][NODE:] + op['b1'][None]
    ow3 = jnp.zeros((HID, 8), jnp.float32).at[:, :3].set(op['W3'])
    ob3 = jnp.zeros((8,), jnp.float32).at[:3].set(op['b3'])
    pack_last = jnp.concatenate([
        op['W1'][:NODE],
        co,
        op['W2'],
        op['b2'][None],
        jnp.zeros((HID, HID), jnp.float32).at[:, :8].set(ow3),
        jnp.zeros((1, HID), jnp.float32).at[0, :8].set(ob3),
    ], axis=0)

    z3 = z.reshape(N // NBLK, 1, NBLK).astype(jnp.int32)
    xp = jnp.zeros((N, 8), jnp.float32).at[:, :3].set(x_t[0])
    d3 = dist.reshape(E // EBLK, 1, EBLK)
    src = edge_index[0].astype(jnp.int32)
    dst = edge_index[1].astype(jnp.int32)

    h, pcur = _embed_call(z3, xp, emb_pad, cw1, c['b1'][None], c['W2'],
                          c['b2'][None], gnn[0]['mW1'][:NODE])
    e = _edge_call(d3, gam, ew1a, ew1b, ep['b1'][None], ep['W2'],
                   ep['b2'][None])

    for l in range(3):
        lp = gnn[l]
        p_src = _gather_call(pcur, src)
        mlo, mhi = _msg_call(p_src, e, lp['mW1'][NODE:NODE + EDGE], cm[l],
                             lp['mW2'], lp['mb2'][None])
        aggl, aggh = _scatter_call(mlo, mhi, dst)
        if l < 2:
            nxt = gnn[l + 1]['mW1'][:NODE]
            call = _node_call_mid
        else:
            nxt = pack_last
            call = _node_call_last
        h, pcur = call(h, aggl, aggh, lp['uW1'][:NODE],
                       lp['uW1'][NODE:NODE + HID], cu[l], lp['uW2'],
                       lp['ub2'][None], lp['ln_g'][None], lp['ln_b'][None],
                       nxt)

    return pcur[:, :3].reshape(1, N, 3)


# speedup vs baseline: 3.2890x; 1.0311x over previous
import functools

import jax
import jax.numpy as jnp
import numpy as np
from jax import lax
from jax.experimental import pallas as pl
from jax.experimental.pallas import tpu as pltpu
from jax.experimental.pallas import tpu_sc as plsc

N = 10000
E = 160000
NODE, EDGE, TIME, COND, HID = 192, 64, 128, 128, 256
CUTOFF, RBF, MAXZ = 5.0, 32, 100

NBLK = 1000
EBLK = 2000
CH = 128
NCH = E // CH
NW = 32
ROWS_PER_TILE = N // 16


def _silu(x):
    return x * (1.0 / (1.0 + jnp.exp(-x)))


def _pack_bf16(x):
    lo = lax.bitcast_convert_type(x[:, :128].astype(jnp.bfloat16),
                                  jnp.uint16).astype(jnp.uint32)
    hi = lax.bitcast_convert_type(x[:, 128:].astype(jnp.bfloat16),
                                  jnp.uint16).astype(jnp.uint32)
    return lo | (hi << 16)


def _unpack_bf16(u):
    lo = lax.bitcast_convert_type((u & 0xFFFF).astype(jnp.uint16),
                                  jnp.bfloat16).astype(jnp.float32)
    hi = lax.bitcast_convert_type((u >> 16).astype(jnp.uint16),
                                  jnp.bfloat16).astype(jnp.float32)
    return jnp.concatenate([lo, hi], axis=1)


def _embed_body(z_ref, x_ref, emb_ref, cw1_ref, cb1_ref, cw2_ref, cb2_ref,
                mw1h_ref, h_ref, p_ref):
    z = z_ref[0, 0, :]
    oh = (z[:, None] == lax.broadcasted_iota(jnp.int32, (1, 128), 1)
          ).astype(jnp.float32)
    h_emb = jnp.dot(oh, emb_ref[...], preferred_element_type=jnp.float32)
    x = x_ref[...]
    hc = jnp.dot(_silu(jnp.dot(x, cw1_ref[...],
                               preferred_element_type=jnp.float32)
                       + cb1_ref[...]),
                 cw2_ref[...], preferred_element_type=jnp.float32) + cb2_ref[...]
    h = h_emb + hc
    h_ref[...] = h
    p_ref[...] = _pack_bf16(
        jnp.dot(h, mw1h_ref[...], preferred_element_type=jnp.float32))


_embed_call = pl.pallas_call(
    _embed_body,
    grid=(N // NBLK,),
    in_specs=[
        pl.BlockSpec((1, 1, NBLK), lambda i: (i, 0, 0)),
        pl.BlockSpec((NBLK, 8), lambda i: (i, 0)),
        pl.BlockSpec((128, NODE), lambda i: (0, 0)),
        pl.BlockSpec((8, NODE), lambda i: (0, 0)),
        pl.BlockSpec((1, NODE), lambda i: (0, 0)),
        pl.BlockSpec((NODE, NODE), lambda i: (0, 0)),
        pl.BlockSpec((1, NODE), lambda i: (0, 0)),
        pl.BlockSpec((NODE, HID), lambda i: (0, 0)),
    ],
    out_specs=[
        pl.BlockSpec((NBLK, NODE), lambda i: (i, 0)),
        pl.BlockSpec((NBLK, 128), lambda i: (i, 0)),
    ],
    out_shape=[
        jax.ShapeDtypeStruct((N, NODE), jnp.float32),
        jax.ShapeDtypeStruct((N, 128), jnp.uint32),
    ],
)


def _edge_body(d_ref, g_ref, ew1a_ref, ew1b_ref, eb1_ref, ew2_ref, eb2_ref,
               e_ref):
    d = jnp.clip(d_ref[0, 0, :], 0.0, CUTOFF)
    dc = d[:, None]
    centers = (lax.broadcasted_iota(jnp.int32, (1, RBF), 1)
               .astype(jnp.float32) * (CUTOFF / (RBF - 1)))
    rbf = jnp.exp(-g_ref[0, 0] * (dc - centers) ** 2)
    f1 = (jnp.dot(rbf, ew1a_ref[...], preferred_element_type=jnp.float32)
          + (dc / CUTOFF) * ew1b_ref[...] + eb1_ref[...])
    e_ref[...] = (jnp.dot(_silu(f1), ew2_ref[...],
                          preferred_element_type=jnp.float32)
                  + eb2_ref[...]).astype(jnp.bfloat16)


_edge_call = pl.pallas_call(
    _edge_body,
    grid=(E // EBLK,),
    in_specs=[
        pl.BlockSpec((1, 1, EBLK), lambda i: (i, 0, 0)),
        pl.BlockSpec((1, 1), lambda i: (0, 0)),
        pl.BlockSpec((RBF, EDGE), lambda i: (0, 0)),
        pl.BlockSpec((1, EDGE), lambda i: (0, 0)),
        pl.BlockSpec((1, EDGE), lambda i: (0, 0)),
        pl.BlockSpec((EDGE, EDGE), lambda i: (0, 0)),
        pl.BlockSpec((1, EDGE), lambda i: (0, 0)),
    ],
    out_specs=pl.BlockSpec((EBLK, EDGE), lambda i: (i, 0)),
    out_shape=jax.ShapeDtypeStruct((E, EDGE), jnp.bfloat16),
)


def _msg_body(ps_ref, e_ref, mw1e_ref, cm_ref, mw2_ref, mb2_ref,
              lo_ref, hi_ref):
    z1 = (_unpack_bf16(ps_ref[...])
          + jnp.dot(e_ref[...], mw1e_ref[...].astype(jnp.bfloat16),
                    preferred_element_type=jnp.float32)
          + cm_ref[...])
    a1 = _silu(z1).astype(jnp.bfloat16)
    msg = _silu(jnp.dot(a1, mw2_ref[...].astype(jnp.bfloat16),
                        preferred_element_type=jnp.float32) + mb2_ref[...])
    lo_ref[...] = msg[:, :128]
    hi_ref[...] = msg[:, 128:]


_msg_call = pl.pallas_call(
    _msg_body,
    grid=(E // EBLK,),
    in_specs=[
        pl.BlockSpec((EBLK, 128), lambda i: (i, 0)),
        pl.BlockSpec((EBLK, EDGE), lambda i: (i, 0)),
        pl.BlockSpec((EDGE, HID), lambda i: (0, 0)),
        pl.BlockSpec((1, HID), lambda i: (0, 0)),
        pl.BlockSpec((HID, HID), lambda i: (0, 0)),
        pl.BlockSpec((1, HID), lambda i: (0, 0)),
    ],
    out_specs=[
        pl.BlockSpec((EBLK, 128), lambda i: (i, 0)),
        pl.BlockSpec((EBLK, 128), lambda i: (i, 0)),
    ],
    out_shape=[
        jax.ShapeDtypeStruct((E, 128), jnp.float32),
        jax.ShapeDtypeStruct((E, 128), jnp.float32),
    ],
)


def _node_body(h_ref, al_ref, ah_ref, uw1h_ref, uw1a_ref, cu_ref, uw2_ref,
               ub2_ref, g_ref, b_ref, nxt_ref, h_out_ref, p_out_ref,
               *, last):
    h = h_ref[...]
    agg = jnp.concatenate([al_ref[...], ah_ref[...]], axis=1)
    u1 = (jnp.dot(h, uw1h_ref[...], preferred_element_type=jnp.float32)
          + jnp.dot(agg, uw1a_ref[...], preferred_element_type=jnp.float32)
          + cu_ref[...])
    dh = jnp.dot(_silu(u1), uw2_ref[...],
                 preferred_element_type=jnp.float32) + ub2_ref[...]
    x = h + dh
    mu = jnp.mean(x, axis=-1, keepdims=True)
    xm = x - mu
    var = jnp.mean(xm * xm, axis=-1, keepdims=True)
    hn = xm * lax.rsqrt(var + 1e-5) * g_ref[...] + b_ref[...]
    h_out_ref[...] = hn
    if last:
        ow1h = nxt_ref[0:NODE, :]
        co = nxt_ref[NODE:NODE + 1, :]
        ow2 = nxt_ref[NODE + 1:NODE + 1 + HID, :]
        ob2 = nxt_ref[NODE + 1 + HID:NODE + 2 + HID, :]
        ow3 = nxt_ref[NODE + 2 + HID:NODE + 2 + 2 * HID, 0:8]
        ob3 = nxt_ref[NODE + 2 + 2 * HID:NODE + 3 + 2 * HID, 0:8]
        a1 = _silu(jnp.dot(hn, ow1h, preferred_element_type=jnp.float32) + co)
        a2 = _silu(jnp.dot(a1, ow2, preferred_element_type=jnp.float32) + ob2)
        p_out_ref[...] = jnp.dot(a2, ow3,
                                 preferred_element_type=jnp.float32) + ob3
    else:
        p_out_ref[...] = _pack_bf16(jnp.dot(hn, nxt_ref[...],
                                            preferred_element_type=jnp.float32))


def _make_node_call(last):
    nxt_shape = (NODE + 2 * HID + 3, HID) if last else (NODE, HID)
    p_shape = (N, 8) if last else (N, 128)
    p_blk = (NBLK, 8) if last else (NBLK, 128)
    p_dtype = jnp.float32 if last else jnp.uint32
    return pl.pallas_call(
        functools.partial(_node_body, last=last),
        grid=(N // NBLK,),
        in_specs=[
            pl.BlockSpec((NBLK, NODE), lambda i: (i, 0)),
            pl.BlockSpec((NBLK, 128), lambda i: (i, 0)),
            pl.BlockSpec((NBLK, 128), lambda i: (i, 0)),
            pl.BlockSpec((NODE, HID), lambda i: (0, 0)),
            pl.BlockSpec((HID, HID), lambda i: (0, 0)),
            pl.BlockSpec((1, HID), lambda i: (0, 0)),
            pl.BlockSpec((HID, NODE), lambda i: (0, 0)),
            pl.BlockSpec((1, NODE), lambda i: (0, 0)),
            pl.BlockSpec((1, NODE), lambda i: (0, 0)),
            pl.BlockSpec((1, NODE), lambda i: (0, 0)),
            pl.BlockSpec(nxt_shape, lambda i: (0, 0)),
        ],
        out_specs=[
            pl.BlockSpec((NBLK, NODE), lambda i: (i, 0)),
            pl.BlockSpec(p_blk, lambda i: (i, 0)),
        ],
        out_shape=[
            jax.ShapeDtypeStruct((N, NODE), jnp.float32),
            jax.ShapeDtypeStruct(p_shape, p_dtype),
        ],
    )


_node_call_mid = _make_node_call(False)
_node_call_last = _make_node_call(True)


_SC_MESH = plsc.VectorSubcoreMesh(core_axis_name="c", subcore_axis_name="s")
_EPT = E // NW
_GFULL = _EPT // CH
_GTAIL = _EPT - _GFULL * CH


def _gather_body(p_hbm, src_hbm, out_hbm, idx_v, r0, r1, tail_v,
                 sg0, sg1, sw0, sw1):
    wid = lax.axis_index("s") * 2 + lax.axis_index("c")
    base = wid * _EPT
    pltpu.sync_copy(src_hbm.at[pl.ds(base, _EPT)], idx_v)
    rows = (r0, r1)
    sg = (sg0, sg1)
    sw = (sw0, sw1)

    def g_dma(k, b):
        return pltpu.make_async_copy(
            p_hbm.at[idx_v.at[pl.ds(k * CH, CH)]], rows[b], sg[b])

    def w_dma(k, b):
        return pltpu.make_async_copy(
            rows[b], out_hbm.at[pl.ds(base + k * CH, CH)], sw[b])

    g_dma(0, 0).start()

    def pair(i, _):
        for par in (0, 1):
            k = 2 * i + par

            @pl.when(k < _GFULL)
            def _():
                g_dma(k, par).wait()
                w_dma(k, par).start()

                @pl.when(k >= 1)
                def _():
                    w_dma(k - 1, 1 - par).wait()

                @pl.when(k + 1 < _GFULL)
                def _():
                    g_dma(k + 1, 1 - par).start()
        return 0

    lax.fori_loop(0, (_GFULL + 1) // 2, pair, 0)
    w_dma(_GFULL - 1, (_GFULL - 1) % 2).wait()
    pltpu.async_copy(p_hbm.at[idx_v.at[pl.ds(_GFULL * CH, _GTAIL)]],
                     tail_v, sg0).wait()
    pltpu.sync_copy(tail_v, out_hbm.at[pl.ds(base + _GFULL * CH, _GTAIL)])


_gather_call = pl.kernel(
    _gather_body,
    out_type=jax.ShapeDtypeStruct((E, 128), jnp.uint32),
    mesh=_SC_MESH,
    scratch_types=[
        pltpu.VMEM((_EPT,), jnp.int32),
        pltpu.VMEM((CH, 128), jnp.uint32),
        pltpu.VMEM((CH, 128), jnp.uint32),
        pltpu.VMEM((_GTAIL, 128), jnp.uint32),
        pltpu.SemaphoreType.DMA,
        pltpu.SemaphoreType.DMA,
        pltpu.SemaphoreType.DMA,
        pltpu.SemaphoreType.DMA,
    ],
)


_SITER = -(-NCH // 16)


def _scatter_body(ml_hbm, mh_hbm, dst_hbm, al_hbm, ah_hbm,
                  i0_v, i1_v, b0_v, b1_v, zb_v, acc_s,
                  si0, si1, sm0, sm1, ss0, ss1):
    c = lax.axis_index("c")
    s = lax.axis_index("s")

    def zrow(i, _):
        for kk in range(8):
            zb_v[i, pl.ds(kk * 16, 16)] = jnp.zeros((16,), jnp.float32)
        return 0

    lax.fori_loop(0, 128, zrow, 0)
    row0 = s * 624
    nfull = jnp.where(s == 15, 5, 4)

    def zcopy(i, _):
        pltpu.sync_copy(zb_v, acc_s.at[pl.ds(row0 + i * 128, 128)])
        return 0

    lax.fori_loop(0, nfull, zcopy, 0)

    @pl.when(s < 15)
    def _():
        pltpu.sync_copy(zb_v.at[pl.ds(0, 112)],
                        acc_s.at[pl.ds(row0 + 512, 112)])

    plsc.subcore_barrier()

    idx = (i0_v, i1_v)
    buf = (b0_v, b1_v)
    si = (si0, si1)
    sm = (sm0, sm1)
    ss = (ss0, ss1)

    def valid(k):
        return (s + k * 16) < NCH

    def reads_start(k, b):
        base = (s + k * 16) * CH
        pltpu.make_async_copy(dst_hbm.at[pl.ds(base, CH)], idx[b],
                              si[b]).start()

        @pl.when(c == 0)
        def _():
            pltpu.make_async_copy(ml_hbm.at[pl.ds(base, CH)], buf[b],
                                  sm[b]).start()

        @pl.when(c == 1)
        def _():
            pltpu.make_async_copy(mh_hbm.at[pl.ds(base, CH)], buf[b],
                                  sm[b]).start()

    def reads_wait(k, b):
        base = (s + k * 16) * CH
        pltpu.make_async_copy(dst_hbm.at[pl.ds(base, CH)], idx[b],
                              si[b]).wait()
        pltpu.make_async_copy(ml_hbm.at[pl.ds(base, CH)], buf[b],
                              sm[b]).wait()

    def sc_dma(b):
        return pltpu.make_async_copy(buf[b], acc_s.at[idx[b]], ss[b])

    @pl.when(valid(0))
    def _():
        reads_start(0, 0)

    def pair(i, _):
        for par in (0, 1):
            k = 2 * i + par

            @pl.when(valid(k))
            def _():
                reads_wait(k, par)
                sc_dma(par).start(add=True)

                @pl.when(k >= 1)
                def _():
                    sc_dma(1 - par).wait()

                @pl.when(valid(k + 1))
                def _():
                    reads_start(k + 1, 1 - par)
        return 0

    lax.fori_loop(0, (_SITER + 1) // 2, pair, 0)
    for k in (_SITER - 2, _SITER - 1):
        @pl.when(valid(k) & ~valid(k + 1))
        def _():
            sc_dma(k % 2).wait()

    plsc.subcore_barrier()

    @pl.when((c == 0) & (s < 15))
    def _():
        pltpu.sync_copy(acc_s.at[pl.ds(row0, 624)],
                        al_hbm.at[pl.ds(row0, 624)])

    @pl.when((c == 1) & (s < 15))
    def _():
        pltpu.sync_copy(acc_s.at[pl.ds(row0, 624)],
                        ah_hbm.at[pl.ds(row0, 624)])

    @pl.when((c == 0) & (s == 15))
    def _():
        pltpu.sync_copy(acc_s.at[pl.ds(15 * 624, 640)],
                        al_hbm.at[pl.ds(15 * 624, 640)])

    @pl.when((c == 1) & (s == 15))
    def _():
        pltpu.sync_copy(acc_s.at[pl.ds(15 * 624, 640)],
                        ah_hbm.at[pl.ds(15 * 624, 640)])


_scatter_call = pl.kernel(
    _scatter_body,
    out_type=[
        jax.ShapeDtypeStruct((N, 128), jnp.float32),
        jax.ShapeDtypeStruct((N, 128), jnp.float32),
    ],
    mesh=_SC_MESH,
    scratch_types=[
        pltpu.VMEM((CH,), jnp.int32),
        pltpu.VMEM((CH,), jnp.int32),
        pltpu.VMEM((CH, 128), jnp.float32),
        pltpu.VMEM((CH, 128), jnp.float32),
        pltpu.VMEM((128, 128), jnp.float32),
        pltpu.VMEM_SHARED((N, 128), jnp.float32),
        pltpu.SemaphoreType.DMA,
        pltpu.SemaphoreType.DMA,
        pltpu.SemaphoreType.DMA,
        pltpu.SemaphoreType.DMA,
        pltpu.SemaphoreType.DMA,
        pltpu.SemaphoreType.DMA,
    ],
)


def kernel(z, x_t, lattice, edge_index, dist, t, y, mask, params):
    p = params
    half = TIME // 2
    freqs = jnp.exp(jnp.linspace(0.0, np.log(10000.0), half) * -1.0)
    targ = t[:, None] * freqs[None, :]
    temb = jnp.concatenate([jnp.sin(targ), jnp.cos(targ)], axis=-1)
    tp = p['time']
    t_emb = _silu(temb @ tp['W1'] + tp['b1']) @ tp['W2'] + tp['b2']
    pp = p['prop']
    y_emb = (_silu(_silu(y @ pp['W1'] + pp['b1']) @ pp['W2'] + pp['b2'])
             @ pp['W3'] + pp['b3'])
    cond = t_emb + y_emb

    emb_pad = jnp.zeros((128, NODE), jnp.float32).at[:MAXZ + 1].set(
        p['atom_emb'])
    c = p['coord']
    cw1 = jnp.zeros((8, NODE), jnp.float32).at[:2].set(c['W1'][:2]).at[2].set(
        c['W1'][2] * 0.1)
    ep = p['edge']
    gam = ep['gamma'].reshape(1, 1)
    ew1a, ew1b = ep['W1'][:RBF], ep['W1'][RBF:RBF + 1]

    gnn = p['gnn']
    cm = [cond @ lp['mW1'][NODE + EDGE:] + lp['mb1'][None] for lp in gnn]
    cu = [cond @ lp['uW1'][NODE + HID:] + lp['ub1'][None] for lp in gnn]
    op = p['out']
    co = cond @ op['W1'][NODE:] + op['b1'][None]
    ow3 = jnp.zeros((HID, 8), jnp.float32).at[:, :3].set(op['W3'])
    ob3 = jnp.zeros((8,), jnp.float32).at[:3].set(op['b3'])
    pack_last = jnp.concatenate([
        op['W1'][:NODE],
        co,
        op['W2'],
        op['b2'][None],
        jnp.zeros((HID, HID), jnp.float32).at[:, :8].set(ow3),
        jnp.zeros((1, HID), jnp.float32).at[0, :8].set(ob3),
    ], axis=0)

    z3 = z.reshape(N // NBLK, 1, NBLK).astype(jnp.int32)
    xp = jnp.zeros((N, 8), jnp.float32).at[:, :3].set(x_t[0])
    d3 = dist.reshape(E // EBLK, 1, EBLK)
    src = edge_index[0].astype(jnp.int32)
    dst = edge_index[1].astype(jnp.int32)

    h, pcur = _embed_call(z3, xp, emb_pad, cw1, c['b1'][None], c['W2'],
                          c['b2'][None], gnn[0]['mW1'][:NODE])
    e = _edge_call(d3, gam, ew1a, ew1b, ep['b1'][None], ep['W2'],
                   ep['b2'][None])

    for l in range(3):
        lp = gnn[l]
        p_src = _gather_call(pcur, src)
        mlo, mhi = _msg_call(p_src, e, lp['mW1'][NODE:NODE + EDGE], cm[l],
                             lp['mW2'], lp['mb2'][None])
        aggl, aggh = _scatter_call(mlo, mhi, dst)
        if l < 2:
            nxt = gnn[l + 1]['mW1'][:NODE]
            call = _node_call_mid
        else:
            nxt = pack_last
            call = _node_call_last
        h, pcur = call(h, aggl, aggh, lp['uW1'][:NODE],
                       lp['uW1'][NODE:NODE + HID], cu[l], lp['uW2'],
                       lp['ub2'][None], lp['ln_g'][None], lp['ln_b'][None],
                       nxt)

    return pcur[:, :3].reshape(1, N, 3)
